# baseline (speedup 1.0000x reference)
import jax
import jax.numpy as jnp
from jax.experimental import pallas as pl

N = 10000
E = 320000
RATIO = 0.5


def _msgmm_kernel(a_ref, b_ref, o_ref):
    o_ref[...] = jnp.dot(a_ref[...], b_ref[...],
                         preferred_element_type=jnp.float32)


def _msgmm(a, b, blk=2000):
    m, k = a.shape
    _, n = b.shape
    return pl.pallas_call(
        _msgmm_kernel,
        grid=(m // blk,),
        in_specs=[pl.BlockSpec((blk, k), lambda i: (i, 0)),
                  pl.BlockSpec((k, n), lambda i: (0, 0))],
        out_specs=pl.BlockSpec((blk, n), lambda i: (i, 0)),
        out_shape=jax.ShapeDtypeStruct((m, n), jnp.float32),
    )(a, b)


def _score_kernel(a_ref, b_ref, c_ref, w_ref, o_ref):
    rep = jnp.concatenate([a_ref[...], b_ref[...]], axis=1)
    o_ref[...] = jnp.dot(rep, w_ref[...],
                         preferred_element_type=jnp.float32) + c_ref[0, 0]


def _score(hs, hd, wl, bl, blk=8000):
    m, k = hs.shape
    return pl.pallas_call(
        _score_kernel,
        grid=(m // blk,),
        in_specs=[pl.BlockSpec((blk, k), lambda i: (i, 0)),
                  pl.BlockSpec((blk, k), lambda i: (i, 0)),
                  pl.BlockSpec((1, 1), lambda i: (0, 0)),
                  pl.BlockSpec((2 * k, 1), lambda i: (0, 0))],
        out_specs=pl.BlockSpec((blk, 1), lambda i: (i, 0)),
        out_shape=jax.ShapeDtypeStruct((m, 1), jnp.float32),
    )(hs, hd, bl.reshape(1, 1), wl)


def _mlp_kernel(agg_ref, x_ref, w_ref, b_ref, o_ref):
    t = agg_ref[...] + x_ref[...]
    y = jnp.dot(t, w_ref[...], preferred_element_type=jnp.float32) + b_ref[...]
    o_ref[...] = jnp.maximum(y, 0.0)


def _mlp(agg, x, w, b, blk=2000):
    m, k = agg.shape
    return pl.pallas_call(
        _mlp_kernel,
        grid=(m // blk,),
        in_specs=[pl.BlockSpec((blk, k), lambda i: (i, 0)),
                  pl.BlockSpec((blk, k), lambda i: (i, 0)),
                  pl.BlockSpec((k, k), lambda i: (0, 0)),
                  pl.BlockSpec((1, k), lambda i: (0, 0))],
        out_specs=pl.BlockSpec((blk, k), lambda i: (i, 0)),
        out_shape=jax.ShapeDtypeStruct((m, k), jnp.float32),
    )(agg, x, w, b.reshape(1, k))


def _gin(x, src, dst, e, We, W, b):
    msg = x[src] + _msgmm(e, We)
    agg = jax.ops.segment_sum(msg, dst, num_segments=N)
    return _mlp(agg, x, W, b)


def kernel(x, edge_index, edge_attr, We1, W1, b1, We2, W2, b2, Wl, bl):
    src = edge_index[0]
    dst = edge_index[1]
    h = _gin(x, src, dst, edge_attr, We1, W1, b1)
    h = _gin(h, src, dst, edge_attr, We2, W2, b2)
    pred_edge_weight = _score(h[src], h[dst], Wl, bl).reshape(-1)
    n_reserve = int(RATIO * E)
    order = jnp.argsort(-pred_edge_weight)
    idx_reserve = order[:n_reserve]
    idx_drop = order[n_reserve:]
    causal_edge_index = edge_index[:, idx_reserve]
    conf_edge_index = edge_index[:, idx_drop]
    causal_edge_weight = pred_edge_weight[idx_reserve]
    conf_edge_weight = -pred_edge_weight[idx_drop]
    causal_edge_attr = edge_attr[idx_reserve]
    conf_edge_attr = edge_attr[idx_drop]
    return (h, causal_edge_index, causal_edge_attr, causal_edge_weight,
            conf_edge_index, conf_edge_attr, conf_edge_weight, pred_edge_weight)
